# baseline (speedup 1.0000x reference)
import functools

import jax
import jax.numpy as jnp
from jax import lax
from jax.experimental import pallas as pl
from jax.experimental.pallas import tpu as pltpu
from jax.experimental.pallas import tpu_sc as plsc

N_NODES = 10000
D_IN = 128
D_H = 64
N_EDGES = 320000

NC = 2
NS = 16
NW = NC * NS

EDGE_BATCH = 128
NBUF = 8
BATCHES_PER_W = 80
NGROUPS = BATCHES_PER_W // NBUF
E_PAD = NW * BATCHES_PER_W * EDGE_BATCH
ACC_ROWS = 10240
ZERO_ROWS = ACC_ROWS // NS
OUT_ROWS = N_NODES // NS

@functools.cache
def _build_segment_sum_sc():
    mesh = plsc.VectorSubcoreMesh(core_axis_name="c", subcore_axis_name="s")
    return functools.partial(
        pl.kernel,
        mesh=mesh,
        compiler_params=pltpu.CompilerParams(use_tc_tiling_on_sc=False),
        out_type=jax.ShapeDtypeStruct((NC, ACC_ROWS, D_H), jnp.float32),
        scratch_types=(
            [
                pltpu.VMEM((BATCHES_PER_W, EDGE_BATCH), jnp.int32),
                pltpu.VMEM((BATCHES_PER_W, EDGE_BATCH), jnp.int32),
                pltpu.VMEM_SHARED((ACC_ROWS, D_H), jnp.float32),
            ]
            + [pltpu.VMEM((EDGE_BATCH, D_H), jnp.float32)] * NBUF
            + [pltpu.SemaphoreType.DMA] * (2 * NBUF)
        ),
    )(_segment_sum_sc_body)


def _segment_sum_sc_body(z_hbm, src_hbm, dst_hbm, zeros_hbm, out_hbm,
                         src_v, dst_v, acc, *rest):
    bufs = rest[:NBUF]
    gsems = rest[NBUF:2 * NBUF]
    ssems = rest[2 * NBUF:3 * NBUF]
    c = lax.axis_index("c")
    s = lax.axis_index("s")
    wid = s * NC + c

    pltpu.sync_copy(zeros_hbm, acc.at[pl.ds(s * ZERO_ROWS, ZERO_ROWS)])
    pltpu.sync_copy(src_hbm.at[wid], src_v)
    pltpu.sync_copy(dst_hbm.at[wid], dst_v)
    plsc.subcore_barrier()

    for b in range(NBUF):
        pltpu.async_copy(z_hbm.at[src_v.at[b]], bufs[b], gsems[b])

    def group_body(g, _):
        base = g * NBUF
        for b in range(NBUF):
            j = base + b
            pltpu.make_async_copy(z_hbm.at[src_v.at[j]], bufs[b],
                                  gsems[b]).wait()
            pltpu.async_copy(bufs[b], acc.at[dst_v.at[j]], ssems[b], add=True)
        for b in range(NBUF):
            j = base + b
            pltpu.make_async_copy(bufs[b], acc.at[dst_v.at[j]],
                                  ssems[b]).wait()
            pltpu.async_copy(z_hbm.at[src_v.at[j + NBUF]], bufs[b], gsems[b])
        return 0

    lax.fori_loop(0, NGROUPS - 1, group_body, 0)
    base = (NGROUPS - 1) * NBUF
    for b in range(NBUF):
        j = base + b
        pltpu.make_async_copy(z_hbm.at[src_v.at[j]], bufs[b], gsems[b]).wait()
        pltpu.async_copy(bufs[b], acc.at[dst_v.at[j]], ssems[b], add=True)
    for b in range(NBUF):
        j = base + b
        pltpu.make_async_copy(bufs[b], acc.at[dst_v.at[j]], ssems[b]).wait()

    plsc.subcore_barrier()
    pltpu.sync_copy(acc.at[pl.ds(s * ZERO_ROWS, ZERO_ROWS)],
                    out_hbm.at[c, pl.ds(s * ZERO_ROWS, ZERO_ROWS)])


def _mm_body(x_ref, w_ref, o_ref):
    o_ref[...] = jnp.dot(x_ref[...], w_ref[...],
                         preferred_element_type=jnp.float32)


def _fuse_body(z_ref, a_ref, b_ref, w_ref, o_ref):
    a = a_ref[0, :N_NODES] + a_ref[1, :N_NODES]
    h = jnp.maximum(z_ref[...] + a + b_ref[...], 0.0)
    o_ref[...] = jnp.dot(h, w_ref[...], preferred_element_type=jnp.float32)


def _head_body(z_ref, a_ref, b2_ref, w3_ref, b3_ref, w4_ref, b4_ref, o_ref):
    a = a_ref[0, :N_NODES] + a_ref[1, :N_NODES]
    h2 = jnp.maximum(z_ref[...] + a + b2_ref[...], 0.0)
    h3 = jnp.maximum(jnp.dot(h2, w3_ref[...],
                             preferred_element_type=jnp.float32) + b3_ref[...],
                     0.0)
    o_ref[...] = jnp.dot(h3, w4_ref[...],
                         preferred_element_type=jnp.float32) + b4_ref[...]


_mm = pl.pallas_call(
    _mm_body, out_shape=jax.ShapeDtypeStruct((N_NODES, D_H), jnp.float32))

_fuse = pl.pallas_call(
    _fuse_body, out_shape=jax.ShapeDtypeStruct((N_NODES, D_H), jnp.float32))

_head = pl.pallas_call(
    _head_body, out_shape=jax.ShapeDtypeStruct((N_NODES, 1), jnp.float32))


def kernel(x, edge_index, batch, W1, b1, W2, b2, W3, b3, W4, b4):
    del batch
    x = x.astype(jnp.float32)
    src = edge_index[0].astype(jnp.int32)
    dst = edge_index[1].astype(jnp.int32)

    pad = E_PAD - N_EDGES
    src_p = jnp.concatenate([src, jnp.zeros((pad,), jnp.int32)])
    dst_p = jnp.concatenate([dst, jnp.full((pad,), N_NODES, jnp.int32)])
    src3 = src_p.reshape(NW, BATCHES_PER_W, EDGE_BATCH)
    dst3 = dst_p.reshape(NW, BATCHES_PER_W, EDGE_BATCH)
    zeros = jnp.zeros((ZERO_ROWS, D_H), jnp.float32)

    b1r = b1.reshape(1, D_H)
    b2r = b2.reshape(1, D_H)
    b3r = b3.reshape(1, 16)
    b4r = b4.reshape(1, 1)

    seg_sum = _build_segment_sum_sc()
    z1 = _mm(x, W1)
    a1 = seg_sum(z1, src3, dst3, zeros)
    z2 = _fuse(z1, a1, b1r, W2)
    a2 = seg_sum(z2, src3, dst3, zeros)
    out = _head(z2, a2, b2r, W3, b3r, W4, b4r)
    return out


# speedup vs baseline: 1.7830x; 1.7830x over previous
import functools

import jax
import jax.numpy as jnp
from jax import lax
from jax.experimental import pallas as pl
from jax.experimental.pallas import tpu as pltpu
from jax.experimental.pallas import tpu_sc as plsc

N_NODES = 10000
D_IN = 128
D_H = 64
N_EDGES = 320000

NC = 2
NS = 16

EDGE_BATCH = 128
T_FAST = 104
T_SLOW = 54
TOTAL_BATCHES = NS * (T_FAST + T_SLOW)
E_PAD = TOTAL_BATCHES * EDGE_BATCH
SLOW_BASE = NS * T_FAST
ACC_ROWS = 10240
ZERO_ROWS = ACC_ROWS // NS


@functools.cache
def _build_segment_sum_sc():
    mesh = plsc.VectorSubcoreMesh(core_axis_name="c", subcore_axis_name="s")
    return functools.partial(
        pl.kernel,
        mesh=mesh,
        compiler_params=pltpu.CompilerParams(use_tc_tiling_on_sc=False),
        out_type=jax.ShapeDtypeStruct((NC, ACC_ROWS, D_H), jnp.float32),
        scratch_types=[
            pltpu.VMEM((T_FAST, EDGE_BATCH), jnp.int32),
            pltpu.VMEM((T_FAST, EDGE_BATCH), jnp.int32),
            pltpu.VMEM((EDGE_BATCH, D_H), jnp.float32),
            pltpu.VMEM((EDGE_BATCH, D_H), jnp.float32),
            pltpu.VMEM_SHARED((ACC_ROWS, D_H), jnp.float32),
            pltpu.SemaphoreType.DMA,
            pltpu.SemaphoreType.DMA,
        ],
    )(_segment_sum_sc_body)


def _segment_sum_sc_body(z_hbm, edges_hbm, zeros_hbm, out_hbm,
                         src_v, dst_v, buf0, buf1, acc, sem0, sem1):
    c = lax.axis_index("c")
    s = lax.axis_index("s")

    pltpu.sync_copy(zeros_hbm, acc.at[pl.ds(s * ZERO_ROWS, ZERO_ROWS)])

    def stage(nb, base):
        pltpu.sync_copy(edges_hbm.at[0, pl.ds(base, nb)], src_v.at[pl.ds(0, nb)])
        pltpu.sync_copy(edges_hbm.at[1, pl.ds(base, nb)], dst_v.at[pl.ds(0, nb)])

    @pl.when(c == 0)
    def _():
        stage(T_FAST, s * T_FAST)

    @pl.when(c == 1)
    def _():
        stage(T_SLOW, SLOW_BASE + s * T_SLOW)

    plsc.subcore_barrier()

    def run(nb):
        npairs = nb // 2
        pltpu.async_copy(z_hbm.at[src_v.at[0]], buf0, sem0)
        pltpu.async_copy(z_hbm.at[src_v.at[1]], buf1, sem1)

        def body(i, _):
            r = 2 * i
            pltpu.make_async_copy(z_hbm.at[src_v.at[r]], buf0, sem0).wait()
            pltpu.sync_copy(buf0, acc.at[dst_v.at[r]], add=True)
            pltpu.async_copy(z_hbm.at[src_v.at[r + 2]], buf0, sem0)
            pltpu.make_async_copy(z_hbm.at[src_v.at[r + 1]], buf1, sem1).wait()
            pltpu.sync_copy(buf1, acc.at[dst_v.at[r + 1]], add=True)
            pltpu.async_copy(z_hbm.at[src_v.at[r + 3]], buf1, sem1)
            return 0

        lax.fori_loop(0, npairs - 1, body, 0)
        r = nb - 2
        pltpu.make_async_copy(z_hbm.at[src_v.at[r]], buf0, sem0).wait()
        pltpu.sync_copy(buf0, acc.at[dst_v.at[r]], add=True)
        pltpu.make_async_copy(z_hbm.at[src_v.at[r + 1]], buf1, sem1).wait()
        pltpu.sync_copy(buf1, acc.at[dst_v.at[r + 1]], add=True)

    @pl.when(c == 0)
    def _():
        run(T_FAST)

    @pl.when(c == 1)
    def _():
        run(T_SLOW)

    plsc.subcore_barrier()
    pltpu.sync_copy(acc.at[pl.ds(s * ZERO_ROWS, ZERO_ROWS)],
                    out_hbm.at[c, pl.ds(s * ZERO_ROWS, ZERO_ROWS)])


def _mm_body(x_ref, w_ref, o_ref):
    o_ref[...] = jnp.dot(x_ref[...], w_ref[...],
                         preferred_element_type=jnp.float32)


def _fuse_body(z_ref, a_ref, b_ref, w_ref, o_ref):
    a = a_ref[0, :N_NODES] + a_ref[1, :N_NODES]
    h = jnp.maximum(z_ref[...] + a + b_ref[...], 0.0)
    o_ref[...] = jnp.dot(h, w_ref[...], preferred_element_type=jnp.float32)


def _head_body(z_ref, a_ref, b2_ref, w3_ref, b3_ref, w4_ref, b4_ref, o_ref):
    a = a_ref[0, :N_NODES] + a_ref[1, :N_NODES]
    h2 = jnp.maximum(z_ref[...] + a + b2_ref[...], 0.0)
    h3 = jnp.maximum(jnp.dot(h2, w3_ref[...],
                             preferred_element_type=jnp.float32) + b3_ref[...],
                     0.0)
    o_ref[...] = jnp.dot(h3, w4_ref[...],
                         preferred_element_type=jnp.float32) + b4_ref[...]


_mm = pl.pallas_call(
    _mm_body, out_shape=jax.ShapeDtypeStruct((N_NODES, D_H), jnp.float32))

_fuse = pl.pallas_call(
    _fuse_body, out_shape=jax.ShapeDtypeStruct((N_NODES, D_H), jnp.float32))

_head = pl.pallas_call(
    _head_body, out_shape=jax.ShapeDtypeStruct((N_NODES, 1), jnp.float32))


def kernel(x, edge_index, batch, W1, b1, W2, b2, W3, b3, W4, b4):
    del batch
    x = x.astype(jnp.float32)

    pad = E_PAD - N_EDGES
    pad_cols = jnp.concatenate(
        [jnp.zeros((1, pad), jnp.int32), jnp.full((1, pad), N_NODES, jnp.int32)])
    edges3 = jnp.concatenate(
        [edge_index.astype(jnp.int32), pad_cols], axis=1
    ).reshape(2, TOTAL_BATCHES, EDGE_BATCH)
    zeros = jnp.zeros((ZERO_ROWS, D_H), jnp.float32)

    b1r = b1.reshape(1, D_H)
    b2r = b2.reshape(1, D_H)
    b3r = b3.reshape(1, 16)
    b4r = b4.reshape(1, 1)

    seg_sum = _build_segment_sum_sc()
    z1 = _mm(x, W1)
    a1 = seg_sum(z1, edges3, zeros)
    z2 = _fuse(z1, a1, b1r, W2)
    a2 = seg_sum(z2, edges3, zeros)
    out = _head(z2, a2, b2r, W3, b3r, W4, b4r)
    return out
